# Initial kernel scaffold; baseline (speedup 1.0000x reference)
import functools

import jax
import jax.numpy as jnp
from jax import lax
from jax.experimental import pallas as pl
from jax.experimental.pallas import tpu as pltpu
from jax.experimental.pallas import tpu_sc as plsc

EMB = 128
N = 100000
NC, NS = 2, 16
NW = NC * NS
BPW = N // NW
CH = 128
NFULL = BPW // CH
TAIL = BPW - NFULL * CH
IPW = NFULL + 1

_mesh = plsc.VectorSubcoreMesh(
    core_axis_name="c", subcore_axis_name="s", num_cores=NC, num_subcores=NS
)


@functools.partial(
    pl.kernel,
    out_type=jax.ShapeDtypeStruct((N, EMB), jnp.float32),
    mesh=_mesh,
    scratch_types=[
        pltpu.VMEM((IPW, CH), jnp.int32),
        pltpu.VMEM((CH, EMB), jnp.float32),
        pltpu.SemaphoreType.DMA,
    ],
)
def _emb_lookup(table_hbm, idx_hbm, out_hbm, idx_v, rows, sem):
    wid = lax.axis_index("s") * NC + lax.axis_index("c")
    base = wid * BPW
    pltpu.sync_copy(idx_hbm.at[wid], idx_v)

    def chunk(j, carry):
        pltpu.async_copy(table_hbm.at[idx_v.at[j]], rows, sem).wait()
        pltpu.sync_copy(rows, out_hbm.at[pl.ds(base + j * CH, CH)])
        return carry

    lax.fori_loop(0, NFULL, chunk, 0)
    pltpu.async_copy(table_hbm.at[idx_v.at[NFULL]], rows, sem).wait()
    pltpu.sync_copy(
        rows.at[pl.ds(0, TAIL)],
        out_hbm.at[pl.ds(base + NFULL * CH, TAIL)],
    )


def kernel(Z, W):
    table = jnp.pad(W, ((1, 0), (0, 0)))
    idx = jnp.pad(
        Z.astype(jnp.int32).reshape(NW, BPW), ((0, 0), (0, IPW * CH - BPW))
    ).reshape(NW, IPW, CH)
    return _emb_lookup(table, idx)


# speedup vs baseline: 1.3942x; 1.3942x over previous
import functools

import jax
import jax.numpy as jnp
from jax import lax
from jax.experimental import pallas as pl
from jax.experimental.pallas import tpu as pltpu
from jax.experimental.pallas import tpu_sc as plsc

EMB = 128
N = 100000
NC, NS = 2, 16
NW = NC * NS
NG = N // 8
GQ, GR = divmod(NG, NW)
CNT_LO = 8 * GQ
CNT_HI = CNT_LO + 8
CH = 128
NFULL = CNT_LO // CH
TAIL = CNT_LO - NFULL * CH

_mesh = plsc.VectorSubcoreMesh(
    core_axis_name="c", subcore_axis_name="s", num_cores=NC, num_subcores=NS
)


@functools.partial(
    pl.kernel,
    out_type=jax.ShapeDtypeStruct((N, EMB), jnp.float32),
    mesh=_mesh,
    scratch_types=[
        pltpu.VMEM((CNT_HI,), jnp.int32),
        pltpu.VMEM((CH, EMB), jnp.float32),
        pltpu.SemaphoreType.DMA,
    ],
)
def _emb_lookup(table_hbm, idx_hbm, out_hbm, idx_v, rows, sem):
    wid = lax.axis_index("s") * NC + lax.axis_index("c")
    base = 8 * (GQ * wid + jnp.minimum(wid, GR))
    has_extra = wid < GR

    pltpu.sync_copy(
        idx_hbm.at[pl.ds(base, CNT_LO)], idx_v.at[pl.ds(0, CNT_LO)]
    )

    @pl.when(has_extra)
    def _():
        pltpu.sync_copy(
            idx_hbm.at[pl.ds(base + CNT_LO, 8)], idx_v.at[pl.ds(CNT_LO, 8)]
        )

    def chunk(j, carry):
        pltpu.async_copy(
            table_hbm.at[idx_v.at[pl.ds(j * CH, CH)]], rows, sem
        ).wait()
        pltpu.sync_copy(rows, out_hbm.at[pl.ds(base + j * CH, CH)])
        return carry

    lax.fori_loop(0, NFULL, chunk, 0)

    t0 = NFULL * CH
    pltpu.async_copy(
        table_hbm.at[idx_v.at[pl.ds(t0, TAIL)]], rows.at[pl.ds(0, TAIL)], sem
    ).wait()
    pltpu.sync_copy(
        rows.at[pl.ds(0, TAIL)], out_hbm.at[pl.ds(base + t0, TAIL)]
    )

    @pl.when(has_extra)
    def _():
        pltpu.async_copy(
            table_hbm.at[idx_v.at[pl.ds(CNT_LO, 8)]], rows.at[pl.ds(0, 8)], sem
        ).wait()
        pltpu.sync_copy(
            rows.at[pl.ds(0, 8)], out_hbm.at[pl.ds(base + CNT_LO, 8)]
        )


def kernel(Z, W):
    table = jnp.pad(W, ((1, 0), (0, 0)))
    return _emb_lookup(table, Z.astype(jnp.int32))


# speedup vs baseline: 1.3955x; 1.0009x over previous
import functools

import jax
import jax.numpy as jnp
from jax import lax
from jax.experimental import pallas as pl
from jax.experimental.pallas import tpu as pltpu
from jax.experimental.pallas import tpu_sc as plsc

EMB = 128
N = 100000
NC, NS = 2, 16
NW = NC * NS
NG = N // 8
GQ, GR = divmod(NG, NW)
CNT_LO = 8 * GQ
CNT_HI = CNT_LO + 8
CH = 128
NFULL = CNT_LO // CH
TAIL = CNT_LO - NFULL * CH
NBUF = 4
NPIPE = NFULL // NBUF

_mesh = plsc.VectorSubcoreMesh(
    core_axis_name="c", subcore_axis_name="s", num_cores=NC, num_subcores=NS
)


@functools.partial(
    pl.kernel,
    out_type=jax.ShapeDtypeStruct((N, EMB), jnp.float32),
    mesh=_mesh,
    scratch_types=[
        pltpu.VMEM((CNT_HI,), jnp.int32),
        [pltpu.VMEM((CH, EMB), jnp.float32) for _ in range(NBUF)],
        [pltpu.SemaphoreType.DMA for _ in range(NBUF)],
    ],
)
def _emb_lookup(table_hbm, idx_hbm, out_hbm, idx_v, rows, sems):
    wid = lax.axis_index("s") * NC + lax.axis_index("c")
    base = 8 * (GQ * wid + jnp.minimum(wid, GR))
    has_extra = wid < GR

    pltpu.sync_copy(
        idx_hbm.at[pl.ds(base, CNT_LO)], idx_v.at[pl.ds(0, CNT_LO)]
    )

    @pl.when(has_extra)
    def _():
        pltpu.sync_copy(
            idx_hbm.at[pl.ds(base + CNT_LO, 8)], idx_v.at[pl.ds(CNT_LO, 8)]
        )

    def fire(j, b):
        pltpu.async_copy(
            table_hbm.at[idx_v.at[pl.ds(j * CH, CH)]], rows[b], sems[b]
        )

    def drain(j, b):
        pltpu.make_async_copy(
            table_hbm.at[idx_v.at[pl.ds(j * CH, CH)]], rows[b], sems[b]
        ).wait()

    for b in range(NBUF):
        fire(b, b)

    def step(p, carry):
        for b in range(NBUF):
            j = p * NBUF + b
            drain(j, b)
            pltpu.sync_copy(rows[b], out_hbm.at[pl.ds(base + j * CH, CH)])

            @pl.when(p < NPIPE - 1)
            def _():
                fire(j + NBUF, b)

        return carry

    lax.fori_loop(0, NPIPE, step, 0)

    t0 = NFULL * CH
    pltpu.async_copy(
        table_hbm.at[idx_v.at[pl.ds(t0, TAIL)]],
        rows[0].at[pl.ds(0, TAIL)],
        sems[0],
    ).wait()
    pltpu.sync_copy(
        rows[0].at[pl.ds(0, TAIL)], out_hbm.at[pl.ds(base + t0, TAIL)]
    )

    @pl.when(has_extra)
    def _():
        pltpu.async_copy(
            table_hbm.at[idx_v.at[pl.ds(CNT_LO, 8)]],
            rows[0].at[pl.ds(0, 8)],
            sems[0],
        ).wait()
        pltpu.sync_copy(
            rows[0].at[pl.ds(0, 8)], out_hbm.at[pl.ds(base + CNT_LO, 8)]
        )


def kernel(Z, W):
    table = jnp.pad(W, ((1, 0), (0, 0)))
    return _emb_lookup(table, Z.astype(jnp.int32))
